# Initial kernel scaffold; baseline (speedup 1.0000x reference)
import jax
import jax.numpy as jnp
from jax.experimental import pallas as pl


def kernel(x, edge_index, edge_attr, u, batch, W1a, b1a, W1b, b1b, W2a, b2a, W2b, b2b):
    raise NotImplementedError("write your pallas kernel here")


# speedup vs baseline: 2.2928x; 2.2928x over previous
import functools

import jax
import jax.numpy as jnp
from jax import lax
from jax.experimental import pallas as pl
from jax.experimental.pallas import tpu as pltpu
from jax.experimental.pallas import tpu_sc as plsc

NC = 2
NS = 16
NW = NC * NS
CH = 80

_mesh = functools.partial(
    plsc.VectorSubcoreMesh,
    core_axis_name="c", subcore_axis_name="s", num_cores=NC, num_subcores=NS,
)


def _xa_body(x_ref, w_ref, b_ref, o_ref):
  o_ref[...] = (
      jnp.dot(x_ref[...], w_ref[...], preferred_element_type=jnp.float32)
      + b_ref[...]
  )


def _xa_call(x, w, b):
  n = x.shape[0]
  bn = 2000
  return pl.pallas_call(
      _xa_body,
      grid=(n // bn,),
      in_specs=[
          pl.BlockSpec((bn, x.shape[1]), lambda i: (i, 0)),
          pl.BlockSpec(w.shape, lambda i: (0, 0)),
          pl.BlockSpec(b.shape, lambda i: (0, 0)),
      ],
      out_specs=pl.BlockSpec((bn, w.shape[1]), lambda i: (i, 0)),
      out_shape=jax.ShapeDtypeStruct((n, w.shape[1]), jnp.float32),
  )(x, w, b)


def _gather_call(xa, row2d):
  nchunk = row2d.shape[0]
  h = xa.shape[1]
  e = nchunk * CH
  cpw = nchunk // NW

  @functools.partial(
      pl.kernel,
      out_type=jax.ShapeDtypeStruct((e, h), jnp.float32),
      mesh=_mesh(),
      scratch_types=[
          pltpu.VMEM((cpw, CH), jnp.int32),
          pltpu.VMEM((2, CH, h), jnp.float32),
          pltpu.SemaphoreType.DMA,
          pltpu.SemaphoreType.DMA,
      ],
      compiler_params=pltpu.CompilerParams(use_tc_tiling_on_sc=False),
  )
  def k(xa_hbm, row_hbm, out_hbm, idx_all, rows_v, gsem, osem):
    w = lax.axis_index("s") * NC + lax.axis_index("c")
    c0 = w * cpw
    pltpu.sync_copy(row_hbm.at[pl.ds(c0, cpw)], idx_all)

    def start_gather(j, b):
      pltpu.async_copy(xa_hbm.at[idx_all.at[j]], rows_v.at[b], gsem)

    def wait_gather(b):
      pltpu.make_async_copy(
          xa_hbm.at[idx_all.at[0]], rows_v.at[b], gsem).wait()

    def start_out(j, b):
      pltpu.async_copy(
          rows_v.at[b], out_hbm.at[pl.ds((c0 + j) * CH, CH)], osem)

    def wait_out(b):
      pltpu.make_async_copy(
          rows_v.at[b], out_hbm.at[pl.ds(0, CH)], osem).wait()

    start_gather(0, 0)

    @pl.loop(0, cpw)
    def _(j):
      b = lax.rem(j, 2)

      wait_gather(b)

      @pl.when(j >= 1)
      def _():
        wait_out(1 - b)

      @pl.when(j + 1 < cpw)
      def _():
        start_gather(j + 1, 1 - b)

      start_out(j, b)

    wait_out(lax.rem(cpw - 1, 2))

  return k(xa, row2d)


def _fused_call(xa, eb, row2d, col2d, zsum, zcnt, ones):
  nchunk = row2d.shape[0]
  n, h = zsum.shape
  cpw = nchunk // NW
  rpt = n // NS
  nk = h // 16

  @functools.partial(
      pl.kernel,
      out_type=(
          jax.ShapeDtypeStruct((NC, n, h), jnp.float32),
          jax.ShapeDtypeStruct((NC, n, 16), jnp.float32),
      ),
      mesh=_mesh(),
      scratch_types=[
          pltpu.VMEM((cpw, CH), jnp.int32),
          pltpu.VMEM((cpw, CH), jnp.int32),
          pltpu.VMEM((2, CH, h), jnp.float32),
          pltpu.VMEM((2, CH, h), jnp.float32),
          pltpu.VMEM((CH, 16), jnp.float32),
          pltpu.VMEM_SHARED((n, h), jnp.float32),
          pltpu.VMEM_SHARED((n, 16), jnp.float32),
          pltpu.SemaphoreType.DMA,
          pltpu.SemaphoreType.DMA,
      ],
      compiler_params=pltpu.CompilerParams(use_tc_tiling_on_sc=False),
  )
  def k(xa_hbm, eb_hbm, row_hbm, col_hbm, zsum_hbm, zcnt_hbm, ones_hbm,
        sums_out, cnt_out,
        ridx, cidx, gx_v, eb_v, ones_v, ssum, scnt, gsem, esem):
    c_ax = lax.axis_index("c")
    s_ax = lax.axis_index("s")
    w = s_ax * NC + c_ax
    c0 = w * cpw
    r0 = s_ax * rpt

    pltpu.sync_copy(zsum_hbm.at[pl.ds(r0, rpt)], ssum.at[pl.ds(r0, rpt)])
    pltpu.sync_copy(zcnt_hbm.at[pl.ds(r0, rpt)], scnt.at[pl.ds(r0, rpt)])
    pltpu.sync_copy(ones_hbm, ones_v)
    pltpu.sync_copy(row_hbm.at[pl.ds(c0, cpw)], ridx)
    pltpu.sync_copy(col_hbm.at[pl.ds(c0, cpw)], cidx)
    plsc.subcore_barrier()

    def start_gather(j, b):
      pltpu.async_copy(xa_hbm.at[ridx.at[j]], gx_v.at[b], gsem)

    def wait_gather(b):
      pltpu.make_async_copy(xa_hbm.at[ridx.at[0]], gx_v.at[b], gsem).wait()

    def start_eb(j, b):
      pltpu.async_copy(
          eb_hbm.at[pl.ds((c0 + j) * CH, CH)], eb_v.at[b], esem)

    def wait_eb(b):
      pltpu.make_async_copy(
          eb_hbm.at[pl.ds(0, CH)], eb_v.at[b], esem).wait()

    start_gather(0, 0)
    start_eb(0, 0)

    @pl.loop(0, cpw)
    def _(j):
      b = lax.rem(j, 2)
      wait_gather(b)
      wait_eb(b)

      @pl.when(j + 1 < cpw)
      def _():
        start_gather(j + 1, 1 - b)
        start_eb(j + 1, 1 - b)

      @pl.loop(0, CH, unroll=4)
      def _(r):
        for kk in range(nk):
          sl = pl.ds(kk * 16, 16)
          t = gx_v[b, r, sl] + eb_v[b, r, sl]
          gx_v[b, r, sl] = jnp.where(
              t > 0, t, jnp.exp(jnp.minimum(t, 0.0)) - 1.0)

      pltpu.sync_copy(gx_v.at[b], ssum.at[cidx.at[j]], add=True)
      pltpu.sync_copy(ones_v, scnt.at[cidx.at[j]], add=True)

    plsc.subcore_barrier()
    pltpu.sync_copy(ssum.at[pl.ds(r0, rpt)],
                    sums_out.at[c_ax].at[pl.ds(r0, rpt)])
    pltpu.sync_copy(scnt.at[pl.ds(r0, rpt)],
                    cnt_out.at[c_ax].at[pl.ds(r0, rpt)])

  return k(xa, eb, row2d, col2d, zsum, zcnt, ones)


def _h_body(gx_ref, ea_ref, w_ref, o_ref):
  t = gx_ref[...] + jnp.dot(
      ea_ref[...], w_ref[...], preferred_element_type=jnp.float32)
  o_ref[...] = jnp.where(t > 0, t, jnp.exp(jnp.minimum(t, 0.0)) - 1.0)


def _eb_body(ea_ref, w_ref, o_ref):
  o_ref[...] = jnp.dot(
      ea_ref[...], w_ref[...], preferred_element_type=jnp.float32)


def _eb_call(ea, w):
  e = ea.shape[0]
  be = 4000
  return pl.pallas_call(
      _eb_body,
      grid=(e // be,),
      in_specs=[
          pl.BlockSpec((be, ea.shape[1]), lambda i: (i, 0)),
          pl.BlockSpec(w.shape, lambda i: (0, 0)),
      ],
      out_specs=pl.BlockSpec((be, w.shape[1]), lambda i: (i, 0)),
      out_shape=jax.ShapeDtypeStruct((e, w.shape[1]), jnp.float32),
  )(ea, w)


def _h_call(gx, ea, w):
  e, h = gx.shape
  be = 4000
  return pl.pallas_call(
      _h_body,
      grid=(e // be,),
      in_specs=[
          pl.BlockSpec((be, h), lambda i: (i, 0)),
          pl.BlockSpec((be, ea.shape[1]), lambda i: (i, 0)),
          pl.BlockSpec(w.shape, lambda i: (0, 0)),
      ],
      out_specs=pl.BlockSpec((be, h), lambda i: (i, 0)),
      out_shape=jax.ShapeDtypeStruct((e, h), jnp.float32),
  )(gx, ea, w)


def _scatter_call(hmat, col2d, zsum, zcnt, ones):
  nchunk = col2d.shape[0]
  n, h = zsum.shape
  cpw = nchunk // NW
  rpt = n // NS

  @functools.partial(
      pl.kernel,
      out_type=(
          jax.ShapeDtypeStruct((NC, n, h), jnp.float32),
          jax.ShapeDtypeStruct((NC, n, 16), jnp.float32),
      ),
      mesh=_mesh(),
      scratch_types=[
          pltpu.VMEM((cpw, CH), jnp.int32),
          pltpu.VMEM((2, CH, h), jnp.float32),
          pltpu.VMEM((CH, 16), jnp.float32),
          pltpu.VMEM_SHARED((n, h), jnp.float32),
          pltpu.VMEM_SHARED((n, 16), jnp.float32),
          pltpu.SemaphoreType.DMA,
      ],
      compiler_params=pltpu.CompilerParams(use_tc_tiling_on_sc=False),
  )
  def k(h_hbm, col_hbm, zsum_hbm, zcnt_hbm, ones_hbm,
        sums_out, cnt_out, idx_all, rows_v, ones_v, ssum, scnt, hsem):
    c_ax = lax.axis_index("c")
    s_ax = lax.axis_index("s")
    w = s_ax * NC + c_ax
    c0 = w * cpw
    r0 = s_ax * rpt

    pltpu.sync_copy(zsum_hbm.at[pl.ds(r0, rpt)], ssum.at[pl.ds(r0, rpt)])
    pltpu.sync_copy(zcnt_hbm.at[pl.ds(r0, rpt)], scnt.at[pl.ds(r0, rpt)])
    pltpu.sync_copy(ones_hbm, ones_v)
    pltpu.sync_copy(col_hbm.at[pl.ds(c0, cpw)], idx_all)
    plsc.subcore_barrier()

    def start_load(j, b):
      pltpu.async_copy(
          h_hbm.at[pl.ds((c0 + j) * CH, CH)], rows_v.at[b], hsem)

    def wait_load(b):
      pltpu.make_async_copy(
          h_hbm.at[pl.ds(0, CH)], rows_v.at[b], hsem).wait()

    start_load(0, 0)

    @pl.loop(0, cpw)
    def _(j):
      b = lax.rem(j, 2)

      wait_load(b)

      @pl.when(j + 1 < cpw)
      def _():
        start_load(j + 1, 1 - b)

      pltpu.sync_copy(rows_v.at[b], ssum.at[idx_all.at[j]], add=True)
      pltpu.sync_copy(ones_v, scnt.at[idx_all.at[j]], add=True)

    plsc.subcore_barrier()
    pltpu.sync_copy(ssum.at[pl.ds(r0, rpt)],
                    sums_out.at[c_ax].at[pl.ds(r0, rpt)])
    pltpu.sync_copy(scnt.at[pl.ds(r0, rpt)],
                    cnt_out.at[c_ax].at[pl.ds(r0, rpt)])

  return k(hmat, col2d, zsum, zcnt, ones)


def _out_body(x_ref, s0_ref, s1_ref, c0_ref, c1_ref, w1b_ref, b1b_ref,
              w2x_ref, w2m_ref, b2a_ref, w2b_ref, b2b_ref, o_ref):
  s = s0_ref[...] + s1_ref[...]
  cnt = (c0_ref[...] + c1_ref[...])[:, :1]
  m = s / jnp.maximum(cnt, 1.0)
  mask = (cnt > 0).astype(jnp.float32)
  mean = (
      jnp.dot(m, w1b_ref[...], preferred_element_type=jnp.float32)
      + b1b_ref[...] * mask
  )
  t = (
      jnp.dot(x_ref[...], w2x_ref[...], preferred_element_type=jnp.float32)
      + jnp.dot(mean, w2m_ref[...], preferred_element_type=jnp.float32)
      + b2a_ref[...]
  )
  t = jnp.where(t > 0, t, jnp.exp(jnp.minimum(t, 0.0)) - 1.0)
  o_ref[...] = (
      jnp.dot(t, w2b_ref[...], preferred_element_type=jnp.float32)
      + b2b_ref[...]
  )


def _out_call(x, s0, s1, c0, c1, w1b, b1b, w2x, w2m, b2a, w2b, b2b):
  n, df = x.shape
  h = s0.shape[1]
  out = w2b.shape[1]
  bn = 2000
  full = lambda a: pl.BlockSpec(a.shape, lambda i: (0,) * a.ndim)
  return pl.pallas_call(
      _out_body,
      grid=(n // bn,),
      in_specs=[
          pl.BlockSpec((bn, df), lambda i: (i, 0)),
          pl.BlockSpec((bn, h), lambda i: (i, 0)),
          pl.BlockSpec((bn, h), lambda i: (i, 0)),
          pl.BlockSpec((bn, 16), lambda i: (i, 0)),
          pl.BlockSpec((bn, 16), lambda i: (i, 0)),
          full(w1b), full(b1b), full(w2x), full(w2m), full(b2a),
          full(w2b), full(b2b),
      ],
      out_specs=pl.BlockSpec((bn, out), lambda i: (i, 0)),
      out_shape=jax.ShapeDtypeStruct((n, out), jnp.float32),
  )(x, s0, s1, c0, c1, w1b, b1b, w2x, w2m, b2a, w2b, b2b)


def kernel(x, edge_index, edge_attr, u, batch,
           W1a, b1a, W1b, b1b, W2a, b2a, W2b, b2b):
  n, df = x.shape
  e, de = edge_attr.shape
  h = W1b.shape[0]
  nchunk = e // CH

  row2d = edge_index[0].astype(jnp.int32).reshape(nchunk, CH)
  col2d = edge_index[1].astype(jnp.int32).reshape(nchunk, CH)

  xa = _xa_call(x, W1a[:df], b1a.reshape(1, h))
  eb = _eb_call(edge_attr, W1a[df:])

  zsum = jnp.zeros((n, h), jnp.float32)
  zcnt = jnp.zeros((n, 16), jnp.float32)
  ones = jnp.ones((CH, 16), jnp.float32)
  sums_p, cnt_p = _fused_call(xa, eb, row2d, col2d, zsum, zcnt, ones)

  return _out_call(
      x, sums_p[0], sums_p[1], cnt_p[0], cnt_p[1],
      W1b, b1b.reshape(1, h), W2a[:df], W2a[df:],
      b2a.reshape(1, h), W2b, b2b.reshape(1, W2b.shape[1]),
  )


# speedup vs baseline: 3.0193x; 1.3169x over previous
import functools

import jax
import jax.numpy as jnp
from jax import lax
from jax.experimental import pallas as pl
from jax.experimental.pallas import tpu as pltpu
from jax.experimental.pallas import tpu_sc as plsc

NC = 2
NS = 16
NW = NC * NS
CH = 80

_mesh = functools.partial(
    plsc.VectorSubcoreMesh,
    core_axis_name="c", subcore_axis_name="s", num_cores=NC, num_subcores=NS,
)


def _xa_body(x_ref, w_ref, b_ref, o_ref):
  o_ref[...] = (
      jnp.dot(x_ref[...], w_ref[...], preferred_element_type=jnp.float32)
      + b_ref[...]
  )


def _xa_call(x, w, b):
  n = x.shape[0]
  bn = 2000
  return pl.pallas_call(
      _xa_body,
      grid=(n // bn,),
      in_specs=[
          pl.BlockSpec((bn, x.shape[1]), lambda i: (i, 0)),
          pl.BlockSpec(w.shape, lambda i: (0, 0)),
          pl.BlockSpec(b.shape, lambda i: (0, 0)),
      ],
      out_specs=pl.BlockSpec((bn, w.shape[1]), lambda i: (i, 0)),
      out_shape=jax.ShapeDtypeStruct((n, w.shape[1]), jnp.float32),
  )(x, w, b)


def _gather_call(xa, row2d):
  nchunk = row2d.shape[0]
  h = xa.shape[1]
  e = nchunk * CH
  cpw = nchunk // NW

  hf = CH // 2

  @functools.partial(
      pl.kernel,
      out_type=jax.ShapeDtypeStruct((e // 2, 128), jnp.float32),
      mesh=_mesh(),
      scratch_types=[
          pltpu.VMEM((cpw, CH), jnp.int32),
          pltpu.VMEM((2, CH, h), jnp.float32),
          pltpu.SemaphoreType.DMA,
          pltpu.SemaphoreType.DMA,
      ],
      compiler_params=pltpu.CompilerParams(use_tc_tiling_on_sc=False),
  )
  def k(xa_hbm, row_hbm, out_hbm, idx_all, rows_v, gsem, osem):
    w = lax.axis_index("s") * NC + lax.axis_index("c")
    c0 = w * cpw
    pltpu.sync_copy(row_hbm.at[pl.ds(c0, cpw)], idx_all)

    def start_gather(j, b):
      pltpu.async_copy(xa_hbm.at[idx_all.at[j]], rows_v.at[b], gsem)

    def wait_gather(b):
      pltpu.make_async_copy(
          xa_hbm.at[idx_all.at[0]], rows_v.at[b], gsem).wait()

    def start_out(j, b):
      r2 = (c0 + j) * hf
      pltpu.async_copy(
          rows_v.at[b].at[pl.ds(0, hf)],
          out_hbm.at[pl.ds(r2, hf), pl.ds(0, h)], osem)
      pltpu.async_copy(
          rows_v.at[b].at[pl.ds(hf, hf)],
          out_hbm.at[pl.ds(r2, hf), pl.ds(h, h)], osem)

    def wait_out(b):
      pltpu.make_async_copy(
          rows_v.at[b].at[pl.ds(0, hf)],
          out_hbm.at[pl.ds(0, hf), pl.ds(0, h)], osem).wait()
      pltpu.make_async_copy(
          rows_v.at[b].at[pl.ds(0, hf)],
          out_hbm.at[pl.ds(0, hf), pl.ds(0, h)], osem).wait()

    start_gather(0, 0)

    @pl.loop(0, cpw)
    def _(j):
      b = lax.rem(j, 2)

      wait_gather(b)

      @pl.when(j >= 1)
      def _():
        wait_out(1 - b)

      @pl.when(j + 1 < cpw)
      def _():
        start_gather(j + 1, 1 - b)

      start_out(j, b)

    wait_out(lax.rem(cpw - 1, 2))

  return k(xa, row2d)


def _h_body(gx_ref, ea_ref, w_ref, o_ref):
  t = gx_ref[...] + jnp.dot(
      ea_ref[...], w_ref[...], preferred_element_type=jnp.float32)
  o_ref[...] = jnp.where(t > 0, t, jnp.exp(jnp.minimum(t, 0.0)) - 1.0)


def _h_call(gx2, ea2, wd):
  e2 = gx2.shape[0]
  be = 2000
  return pl.pallas_call(
      _h_body,
      grid=(e2 // be,),
      in_specs=[
          pl.BlockSpec((be, 128), lambda i: (i, 0)),
          pl.BlockSpec((be, ea2.shape[1]), lambda i: (i, 0)),
          pl.BlockSpec(wd.shape, lambda i: (0, 0)),
      ],
      out_specs=pl.BlockSpec((be, 128), lambda i: (i, 0)),
      out_shape=jax.ShapeDtypeStruct((e2, 128), jnp.float32),
  )(gx2, ea2, wd)


def _scatter_call(hmat, col2d, zsum, zcnt, ones):
  nchunk = col2d.shape[0]
  n, h = zsum.shape
  cpw = nchunk // NW
  rpt = n // NS

  @functools.partial(
      pl.kernel,
      out_type=(
          jax.ShapeDtypeStruct((NC, n, h), jnp.float32),
          jax.ShapeDtypeStruct((NC, n, 16), jnp.float32),
      ),
      mesh=_mesh(),
      scratch_types=[
          pltpu.VMEM((cpw, CH), jnp.int32),
          pltpu.VMEM((2, CH, h), jnp.float32),
          pltpu.VMEM((CH, 16), jnp.float32),
          pltpu.VMEM_SHARED((n, h), jnp.float32),
          pltpu.VMEM_SHARED((n, 16), jnp.float32),
          pltpu.SemaphoreType.DMA,
      ],
      compiler_params=pltpu.CompilerParams(use_tc_tiling_on_sc=False),
  )
  def k(h_hbm, col_hbm, zsum_hbm, zcnt_hbm, ones_hbm,
        sums_out, cnt_out, idx_all, rows_v, ones_v, ssum, scnt, hsem):
    c_ax = lax.axis_index("c")
    s_ax = lax.axis_index("s")
    w = s_ax * NC + c_ax
    c0 = w * cpw
    r0 = s_ax * rpt

    pltpu.sync_copy(zsum_hbm.at[pl.ds(r0, rpt)], ssum.at[pl.ds(r0, rpt)])
    pltpu.sync_copy(zcnt_hbm.at[pl.ds(r0, rpt)], scnt.at[pl.ds(r0, rpt)])
    pltpu.sync_copy(ones_hbm, ones_v)
    pltpu.sync_copy(col_hbm.at[pl.ds(c0, cpw)], idx_all)
    plsc.subcore_barrier()

    hf = CH // 2

    def start_load(j, b):
      r2 = (c0 + j) * hf
      pltpu.async_copy(
          h_hbm.at[pl.ds(r2, hf), pl.ds(0, h)],
          rows_v.at[b].at[pl.ds(0, hf)], hsem)
      pltpu.async_copy(
          h_hbm.at[pl.ds(r2, hf), pl.ds(h, h)],
          rows_v.at[b].at[pl.ds(hf, hf)], hsem)

    def wait_load(b):
      pltpu.make_async_copy(
          h_hbm.at[pl.ds(0, hf), pl.ds(0, h)],
          rows_v.at[b].at[pl.ds(0, hf)], hsem).wait()
      pltpu.make_async_copy(
          h_hbm.at[pl.ds(0, hf), pl.ds(0, h)],
          rows_v.at[b].at[pl.ds(0, hf)], hsem).wait()

    start_load(0, 0)

    @pl.loop(0, cpw)
    def _(j):
      b = lax.rem(j, 2)

      wait_load(b)

      @pl.when(j + 1 < cpw)
      def _():
        start_load(j + 1, 1 - b)

      pltpu.sync_copy(rows_v.at[b], ssum.at[idx_all.at[j]], add=True)
      pltpu.sync_copy(ones_v, scnt.at[idx_all.at[j]], add=True)

    plsc.subcore_barrier()
    pltpu.sync_copy(ssum.at[pl.ds(r0, rpt)],
                    sums_out.at[c_ax].at[pl.ds(r0, rpt)])
    pltpu.sync_copy(scnt.at[pl.ds(r0, rpt)],
                    cnt_out.at[c_ax].at[pl.ds(r0, rpt)])

  return k(hmat, col2d, zsum, zcnt, ones)


def _out_body(x_ref, s0_ref, s1_ref, c0_ref, c1_ref, w1b_ref, b1b_ref,
              w2x_ref, w2m_ref, b2a_ref, w2b_ref, b2b_ref, o_ref):
  s = s0_ref[...] + s1_ref[...]
  cnt = (c0_ref[...] + c1_ref[...])[:, :1]
  m = s / jnp.maximum(cnt, 1.0)
  mask = (cnt > 0).astype(jnp.float32)
  mean = (
      jnp.dot(m, w1b_ref[...], preferred_element_type=jnp.float32)
      + b1b_ref[...] * mask
  )
  t = (
      jnp.dot(x_ref[...], w2x_ref[...], preferred_element_type=jnp.float32)
      + jnp.dot(mean, w2m_ref[...], preferred_element_type=jnp.float32)
      + b2a_ref[...]
  )
  t = jnp.where(t > 0, t, jnp.exp(jnp.minimum(t, 0.0)) - 1.0)
  o_ref[...] = (
      jnp.dot(t, w2b_ref[...], preferred_element_type=jnp.float32)
      + b2b_ref[...]
  )


def _out_call(x, s0, s1, c0, c1, w1b, b1b, w2x, w2m, b2a, w2b, b2b):
  n, df = x.shape
  h = s0.shape[1]
  out = w2b.shape[1]
  bn = 2000
  full = lambda a: pl.BlockSpec(a.shape, lambda i: (0,) * a.ndim)
  return pl.pallas_call(
      _out_body,
      grid=(n // bn,),
      in_specs=[
          pl.BlockSpec((bn, df), lambda i: (i, 0)),
          pl.BlockSpec((bn, h), lambda i: (i, 0)),
          pl.BlockSpec((bn, h), lambda i: (i, 0)),
          pl.BlockSpec((bn, 16), lambda i: (i, 0)),
          pl.BlockSpec((bn, 16), lambda i: (i, 0)),
          full(w1b), full(b1b), full(w2x), full(w2m), full(b2a),
          full(w2b), full(b2b),
      ],
      out_specs=pl.BlockSpec((bn, out), lambda i: (i, 0)),
      out_shape=jax.ShapeDtypeStruct((n, out), jnp.float32),
  )(x, s0, s1, c0, c1, w1b, b1b, w2x, w2m, b2a, w2b, b2b)


def kernel(x, edge_index, edge_attr, u, batch,
           W1a, b1a, W1b, b1b, W2a, b2a, W2b, b2b):
  n, df = x.shape
  e, de = edge_attr.shape
  h = W1b.shape[0]
  nchunk = e // CH

  def perm(a):
    a = a.astype(jnp.int32).reshape(nchunk, CH // 2, 2)
    return a.transpose(0, 2, 1).reshape(nchunk, CH)

  row2d = perm(edge_index[0])
  col2d = perm(edge_index[1])
  ea2 = edge_attr.reshape(e // 2, 2 * de)
  w1ae = W1a[df:]
  wd = jnp.zeros((2 * de, 2 * h), jnp.float32)
  wd = wd.at[:de, :h].set(w1ae).at[de:, h:].set(w1ae)

  xa = _xa_call(x, W1a[:df], b1a.reshape(1, h))
  gx2 = _gather_call(xa, row2d)
  hmat = _h_call(gx2, ea2, wd)

  zsum = jnp.zeros((n, h), jnp.float32)
  zcnt = jnp.zeros((n, 16), jnp.float32)
  ones = jnp.ones((CH, 16), jnp.float32)
  sums_p, cnt_p = _scatter_call(hmat, col2d, zsum, zcnt, ones)

  return _out_call(
      x, sums_p[0], sums_p[1], cnt_p[0], cnt_p[1],
      W1b, b1b.reshape(1, h), W2a[:df], W2a[df:],
      b2a.reshape(1, h), W2b, b2b.reshape(1, W2b.shape[1]),
  )


# speedup vs baseline: 4.2999x; 1.4241x over previous
import functools

import jax
import jax.numpy as jnp
from jax import lax
from jax.experimental import pallas as pl
from jax.experimental.pallas import tpu as pltpu
from jax.experimental.pallas import tpu_sc as plsc

NC = 2
NS = 16
NW = NC * NS
CH = 80

_mesh = functools.partial(
    plsc.VectorSubcoreMesh,
    core_axis_name="c", subcore_axis_name="s", num_cores=NC, num_subcores=NS,
)


def _xa_body(x_ref, w_ref, b_ref, o_ref):
  o_ref[...] = (
      jnp.dot(x_ref[...], w_ref[...], preferred_element_type=jnp.float32)
      + b_ref[...]
  )


def _xa_call(x, w, b):
  n = x.shape[0]
  bn = 2000
  return pl.pallas_call(
      _xa_body,
      grid=(n // bn,),
      in_specs=[
          pl.BlockSpec((bn, x.shape[1]), lambda i: (i, 0)),
          pl.BlockSpec(w.shape, lambda i: (0, 0)),
          pl.BlockSpec(b.shape, lambda i: (0, 0)),
      ],
      out_specs=pl.BlockSpec((bn, w.shape[1]), lambda i: (i, 0)),
      out_shape=jax.ShapeDtypeStruct((n, w.shape[1]), jnp.float32),
  )(x, w, b)


def _gather_call(xa128, row_flat, h):
  e = row_flat.shape[0]
  nchunk = e // CH
  cpw = nchunk // NW

  hf = CH // 2

  @functools.partial(
      pl.kernel,
      out_type=jax.ShapeDtypeStruct((e, 128), jnp.float32),
      mesh=_mesh(),
      scratch_types=[
          pltpu.VMEM((cpw * CH,), jnp.int32),
          pltpu.VMEM((2, CH, 128), jnp.float32),
          pltpu.SemaphoreType.DMA,
          pltpu.SemaphoreType.DMA,
      ],
  )
  def k(xa_hbm, row_hbm, out_hbm, idx_all, rows_v, gsem, osem):
    w = lax.axis_index("s") * NC + lax.axis_index("c")
    c0 = w * cpw
    pltpu.sync_copy(row_hbm.at[pl.ds(c0 * CH, cpw * CH)], idx_all)

    def start_gather(j, b):
      pltpu.async_copy(
          xa_hbm.at[idx_all.at[pl.ds(j * CH, CH)]], rows_v.at[b], gsem)

    def wait_gather(b):
      pltpu.make_async_copy(
          xa_hbm.at[idx_all.at[pl.ds(0, CH)]], rows_v.at[b], gsem).wait()

    def start_out(j, b):
      pltpu.async_copy(
          rows_v.at[b], out_hbm.at[pl.ds((c0 + j) * CH, CH)], osem)

    def wait_out(b):
      pltpu.make_async_copy(
          rows_v.at[b], out_hbm.at[pl.ds(0, CH)], osem).wait()

    start_gather(0, 0)

    @pl.loop(0, cpw)
    def _(j):
      b = lax.rem(j, 2)

      wait_gather(b)

      @pl.when(j >= 1)
      def _():
        wait_out(1 - b)

      @pl.when(j + 1 < cpw)
      def _():
        start_gather(j + 1, 1 - b)

      start_out(j, b)

    wait_out(lax.rem(cpw - 1, 2))

  return k(xa128, row_flat)


def _h_body(gx_ref, ea_ref, w_ref, o_ref, *, h, bc, hf):
  de = ea_ref.shape[1]
  eb = jnp.dot(ea_ref[...], w_ref[...], preferred_element_type=jnp.float32)
  t = gx_ref[..., :h] + eb
  res = jnp.where(t > 0, t, jnp.exp(jnp.minimum(t, 0.0)) - 1.0)
  r3 = res.reshape(bc, 2, hf, h)
  o_ref[...] = jnp.concatenate([r3[:, 0], r3[:, 1]], axis=-1).reshape(
      bc * hf, 2 * h)


def _h_call(gx, ea, w):
  e = gx.shape[0]
  h = w.shape[1]
  hf = CH // 2
  bc = 50
  be = bc * CH
  body = functools.partial(_h_body, h=h, bc=bc, hf=hf)
  return pl.pallas_call(
      body,
      grid=(e // be,),
      in_specs=[
          pl.BlockSpec((be, 128), lambda i: (i, 0)),
          pl.BlockSpec((be, ea.shape[1]), lambda i: (i, 0)),
          pl.BlockSpec(w.shape, lambda i: (0, 0)),
      ],
      out_specs=pl.BlockSpec((be // 2, 128), lambda i: (i, 0)),
      out_shape=jax.ShapeDtypeStruct((e // 2, 128), jnp.float32),
  )(gx, ea, w)


def _scatter_call(hmat, col2d, zsum, zcnt, ones):
  nchunk = col2d.shape[0]
  n, h = zsum.shape
  cpw = nchunk // NW
  rpt = n // NS

  @functools.partial(
      pl.kernel,
      out_type=(
          jax.ShapeDtypeStruct((NC, n, h), jnp.float32),
          jax.ShapeDtypeStruct((NC, n, 16), jnp.float32),
      ),
      mesh=_mesh(),
      scratch_types=[
          pltpu.VMEM((cpw, CH), jnp.int32),
          pltpu.VMEM((2, CH, h), jnp.float32),
          pltpu.VMEM((CH, 16), jnp.float32),
          pltpu.VMEM_SHARED((n, h), jnp.float32),
          pltpu.VMEM_SHARED((n, 16), jnp.float32),
          pltpu.SemaphoreType.DMA,
      ],
      compiler_params=pltpu.CompilerParams(use_tc_tiling_on_sc=False),
  )
  def k(h_hbm, col_hbm, zsum_hbm, zcnt_hbm, ones_hbm,
        sums_out, cnt_out, idx_all, rows_v, ones_v, ssum, scnt, hsem):
    c_ax = lax.axis_index("c")
    s_ax = lax.axis_index("s")
    w = s_ax * NC + c_ax
    c0 = w * cpw
    r0 = s_ax * rpt

    pltpu.sync_copy(zsum_hbm.at[pl.ds(r0, rpt)], ssum.at[pl.ds(r0, rpt)])
    pltpu.sync_copy(zcnt_hbm.at[pl.ds(r0, rpt)], scnt.at[pl.ds(r0, rpt)])
    pltpu.sync_copy(ones_hbm, ones_v)
    pltpu.sync_copy(col_hbm.at[pl.ds(c0, cpw)], idx_all)
    plsc.subcore_barrier()

    hf = CH // 2

    def start_load(j, b):
      r2 = (c0 + j) * hf
      pltpu.async_copy(
          h_hbm.at[pl.ds(r2, hf), pl.ds(0, h)],
          rows_v.at[b].at[pl.ds(0, hf)], hsem)
      pltpu.async_copy(
          h_hbm.at[pl.ds(r2, hf), pl.ds(h, h)],
          rows_v.at[b].at[pl.ds(hf, hf)], hsem)

    def wait_load(b):
      pltpu.make_async_copy(
          h_hbm.at[pl.ds(0, hf), pl.ds(0, h)],
          rows_v.at[b].at[pl.ds(0, hf)], hsem).wait()
      pltpu.make_async_copy(
          h_hbm.at[pl.ds(0, hf), pl.ds(0, h)],
          rows_v.at[b].at[pl.ds(0, hf)], hsem).wait()

    start_load(0, 0)

    @pl.loop(0, cpw)
    def _(j):
      b = lax.rem(j, 2)

      wait_load(b)

      @pl.when(j + 1 < cpw)
      def _():
        start_load(j + 1, 1 - b)

      pltpu.sync_copy(rows_v.at[b], ssum.at[idx_all.at[j]], add=True)
      pltpu.sync_copy(ones_v, scnt.at[idx_all.at[j]], add=True)

    plsc.subcore_barrier()
    pltpu.sync_copy(ssum.at[pl.ds(r0, rpt)],
                    sums_out.at[c_ax].at[pl.ds(r0, rpt)])
    pltpu.sync_copy(scnt.at[pl.ds(r0, rpt)],
                    cnt_out.at[c_ax].at[pl.ds(r0, rpt)])

  return k(hmat, col2d, zsum, zcnt, ones)


def _out_body(x_ref, s0_ref, s1_ref, c0_ref, c1_ref, w1b_ref, b1b_ref,
              w2x_ref, w2m_ref, b2a_ref, w2b_ref, b2b_ref, o_ref):
  s = s0_ref[...] + s1_ref[...]
  cnt = (c0_ref[...] + c1_ref[...])[:, :1]
  m = s / jnp.maximum(cnt, 1.0)
  mask = (cnt > 0).astype(jnp.float32)
  mean = (
      jnp.dot(m, w1b_ref[...], preferred_element_type=jnp.float32)
      + b1b_ref[...] * mask
  )
  t = (
      jnp.dot(x_ref[...], w2x_ref[...], preferred_element_type=jnp.float32)
      + jnp.dot(mean, w2m_ref[...], preferred_element_type=jnp.float32)
      + b2a_ref[...]
  )
  t = jnp.where(t > 0, t, jnp.exp(jnp.minimum(t, 0.0)) - 1.0)
  o_ref[...] = (
      jnp.dot(t, w2b_ref[...], preferred_element_type=jnp.float32)
      + b2b_ref[...]
  )


def _out_call(x, s0, s1, c0, c1, w1b, b1b, w2x, w2m, b2a, w2b, b2b):
  n, df = x.shape
  h = s0.shape[1]
  out = w2b.shape[1]
  bn = 2000
  full = lambda a: pl.BlockSpec(a.shape, lambda i: (0,) * a.ndim)
  return pl.pallas_call(
      _out_body,
      grid=(n // bn,),
      in_specs=[
          pl.BlockSpec((bn, df), lambda i: (i, 0)),
          pl.BlockSpec((bn, h), lambda i: (i, 0)),
          pl.BlockSpec((bn, h), lambda i: (i, 0)),
          pl.BlockSpec((bn, 16), lambda i: (i, 0)),
          pl.BlockSpec((bn, 16), lambda i: (i, 0)),
          full(w1b), full(b1b), full(w2x), full(w2m), full(b2a),
          full(w2b), full(b2b),
      ],
      out_specs=pl.BlockSpec((bn, out), lambda i: (i, 0)),
      out_shape=jax.ShapeDtypeStruct((n, out), jnp.float32),
  )(x, s0, s1, c0, c1, w1b, b1b, w2x, w2m, b2a, w2b, b2b)


def kernel(x, edge_index, edge_attr, u, batch,
           W1a, b1a, W1b, b1b, W2a, b2a, W2b, b2b):
  n, df = x.shape
  e, de = edge_attr.shape
  h = W1b.shape[0]
  nchunk = e // CH

  row_flat = edge_index[0].astype(jnp.int32)
  col2d = edge_index[1].astype(jnp.int32).reshape(nchunk, CH)

  w1ax128 = jnp.pad(W1a[:df], ((0, 0), (0, 128 - h)))
  b1a128 = jnp.pad(b1a, (0, 128 - h)).reshape(1, 128)
  xa128 = _xa_call(x, w1ax128, b1a128)
  gx = _gather_call(xa128, row_flat, h)
  hmat = _h_call(gx, edge_attr, W1a[df:])

  zsum = jnp.zeros((n, h), jnp.float32)
  zcnt = jnp.zeros((n, 16), jnp.float32)
  ones = jnp.ones((CH, 16), jnp.float32)
  sums_p, cnt_p = _scatter_call(hmat, col2d, zsum, zcnt, ones)

  return _out_call(
      x, sums_p[0], sums_p[1], cnt_p[0], cnt_p[1],
      W1b, b1b.reshape(1, h), W2a[:df], W2a[df:],
      b2a.reshape(1, h), W2b, b2b.reshape(1, W2b.shape[1]),
  )


# speedup vs baseline: 5.0472x; 1.1738x over previous
import functools

import jax
import jax.numpy as jnp
from jax import lax
from jax.experimental import pallas as pl
from jax.experimental.pallas import tpu as pltpu
from jax.experimental.pallas import tpu_sc as plsc

NC = 2
NS = 16
NW = NC * NS
CH = 80

_mesh = functools.partial(
    plsc.VectorSubcoreMesh,
    core_axis_name="c", subcore_axis_name="s", num_cores=NC, num_subcores=NS,
)


def _xa_body(x_ref, w_ref, b_ref, o_ref):
  o_ref[...] = (
      jnp.dot(x_ref[...], w_ref[...], preferred_element_type=jnp.float32)
      + b_ref[...]
  )


def _xa_call(x, w, b):
  n = x.shape[0]
  bn = 2000
  return pl.pallas_call(
      _xa_body,
      grid=(n // bn,),
      in_specs=[
          pl.BlockSpec((bn, x.shape[1]), lambda i: (i, 0)),
          pl.BlockSpec(w.shape, lambda i: (0, 0)),
          pl.BlockSpec(b.shape, lambda i: (0, 0)),
      ],
      out_specs=pl.BlockSpec((bn, w.shape[1]), lambda i: (i, 0)),
      out_shape=jax.ShapeDtypeStruct((n, w.shape[1]), jnp.float32),
  )(x, w, b)


def _gather_call(xa128, row_flat, h):
  e = row_flat.shape[0]
  nchunk = e // CH
  cpw = nchunk // NW

  hf = CH // 2

  @functools.partial(
      pl.kernel,
      out_type=jax.ShapeDtypeStruct((e, 128), jnp.float32),
      mesh=_mesh(),
      scratch_types=[
          pltpu.VMEM((cpw * CH,), jnp.int32),
          pltpu.VMEM((4, CH, 128), jnp.float32),
          pltpu.SemaphoreType.DMA,
          pltpu.SemaphoreType.DMA,
      ],
  )
  def k(xa_hbm, row_hbm, out_hbm, idx_all, rows_v, gsem, osem):
    w = lax.axis_index("s") * NC + lax.axis_index("c")
    c0 = w * cpw
    pltpu.sync_copy(row_hbm.at[pl.ds(c0 * CH, cpw * CH)], idx_all)

    def start_gather(j, b):
      pltpu.async_copy(
          xa_hbm.at[idx_all.at[pl.ds(j * CH, CH)]], rows_v.at[b], gsem)

    def wait_gather(b):
      pltpu.make_async_copy(
          xa_hbm.at[idx_all.at[pl.ds(0, CH)]], rows_v.at[b], gsem).wait()

    def start_out(j, b):
      pltpu.async_copy(
          rows_v.at[b], out_hbm.at[pl.ds((c0 + j) * CH, CH)], osem)

    def wait_out(b):
      pltpu.make_async_copy(
          rows_v.at[b], out_hbm.at[pl.ds(0, CH)], osem).wait()

    start_gather(0, 0)
    start_gather(1, 1)
    start_gather(2, 2)

    @pl.loop(0, cpw)
    def _(j):
      b = lax.rem(j, 4)

      wait_gather(b)

      @pl.when(j >= 1)
      def _():
        wait_out(b)

      @pl.when(j + 3 < cpw)
      def _():
        start_gather(j + 3, lax.rem(j + 3, 4))

      start_out(j, b)

    wait_out(lax.rem(cpw - 1, 4))

  return k(xa128, row_flat)


def _h_body(gx_ref, ea_ref, w_ref, o_ref, *, h, bc, hf):
  de = ea_ref.shape[1]
  eb = jnp.dot(ea_ref[...], w_ref[...], preferred_element_type=jnp.float32)
  t = gx_ref[..., :h] + eb
  res = jnp.where(t > 0, t, jnp.exp(jnp.minimum(t, 0.0)) - 1.0)
  r3 = res.reshape(bc, 2, hf, h)
  o_ref[...] = jnp.concatenate([r3[:, 0], r3[:, 1]], axis=-1).reshape(
      bc * hf, 2 * h)


def _h_call(gx, ea, w):
  e = gx.shape[0]
  h = w.shape[1]
  hf = CH // 2
  bc = 50
  be = bc * CH
  body = functools.partial(_h_body, h=h, bc=bc, hf=hf)
  return pl.pallas_call(
      body,
      grid=(e // be,),
      in_specs=[
          pl.BlockSpec((be, 128), lambda i: (i, 0)),
          pl.BlockSpec((be, ea.shape[1]), lambda i: (i, 0)),
          pl.BlockSpec(w.shape, lambda i: (0, 0)),
      ],
      out_specs=pl.BlockSpec((be // 2, 128), lambda i: (i, 0)),
      out_shape=jax.ShapeDtypeStruct((e // 2, 128), jnp.float32),
  )(gx, ea, w)


def _scatter_call(hmat, col2d, zsum, zcnt, ones):
  nchunk = col2d.shape[0]
  n, h = zsum.shape
  cpw = nchunk // NW
  rpt = n // NS

  @functools.partial(
      pl.kernel,
      out_type=(
          jax.ShapeDtypeStruct((NC, n, h), jnp.float32),
          jax.ShapeDtypeStruct((NC, n, 16), jnp.float32),
      ),
      mesh=_mesh(),
      scratch_types=[
          pltpu.VMEM((cpw, CH), jnp.int32),
          pltpu.VMEM((4, CH, h), jnp.float32),
          pltpu.VMEM((CH, 16), jnp.float32),
          pltpu.VMEM_SHARED((n, h), jnp.float32),
          pltpu.VMEM_SHARED((n, 16), jnp.float32),
          pltpu.SemaphoreType.DMA,
          pltpu.SemaphoreType.DMA,
      ],
      compiler_params=pltpu.CompilerParams(use_tc_tiling_on_sc=False),
  )
  def k(h_hbm, col_hbm, zsum_hbm, zcnt_hbm, ones_hbm,
        sums_out, cnt_out, idx_all, rows_v, ones_v, ssum, scnt, hsem,
        ssem):
    c_ax = lax.axis_index("c")
    s_ax = lax.axis_index("s")
    w = s_ax * NC + c_ax
    c0 = w * cpw
    r0 = s_ax * rpt

    pltpu.sync_copy(zsum_hbm.at[pl.ds(r0, rpt)], ssum.at[pl.ds(r0, rpt)])
    pltpu.sync_copy(zcnt_hbm.at[pl.ds(r0, rpt)], scnt.at[pl.ds(r0, rpt)])
    pltpu.sync_copy(ones_hbm, ones_v)
    pltpu.sync_copy(col_hbm.at[pl.ds(c0, cpw)], idx_all)
    plsc.subcore_barrier()

    hf = CH // 2

    def start_load(j, b):
      r2 = (c0 + j) * hf
      pltpu.async_copy(
          h_hbm.at[pl.ds(r2, hf), pl.ds(0, h)],
          rows_v.at[b].at[pl.ds(0, hf)], hsem)
      pltpu.async_copy(
          h_hbm.at[pl.ds(r2, hf), pl.ds(h, h)],
          rows_v.at[b].at[pl.ds(hf, hf)], hsem)

    def wait_load(b):
      pltpu.make_async_copy(
          h_hbm.at[pl.ds(0, hf), pl.ds(0, h)],
          rows_v.at[b].at[pl.ds(0, hf)], hsem).wait()
      pltpu.make_async_copy(
          h_hbm.at[pl.ds(0, hf), pl.ds(0, h)],
          rows_v.at[b].at[pl.ds(0, hf)], hsem).wait()

    def wait_scatter(j, b):
      pltpu.make_async_copy(
          rows_v.at[b], ssum.at[idx_all.at[j]], ssem).wait()
      pltpu.make_async_copy(
          ones_v, scnt.at[idx_all.at[j]], ssem).wait()

    start_load(0, 0)
    start_load(1, 1)
    start_load(2, 2)

    @pl.loop(0, cpw)
    def _(j):
      b = lax.rem(j, 4)

      wait_load(b)

      @pl.when(j >= 1)
      def _():
        wait_scatter(j - 1, lax.rem(j - 1, 4))

      @pl.when(j + 3 < cpw)
      def _():
        start_load(j + 3, lax.rem(j + 3, 4))

      pltpu.async_copy(rows_v.at[b], ssum.at[idx_all.at[j]], ssem,
                       add=True)
      pltpu.async_copy(ones_v, scnt.at[idx_all.at[j]], ssem, add=True)

    wait_scatter(cpw - 1, lax.rem(cpw - 1, 4))
    plsc.subcore_barrier()
    pltpu.sync_copy(ssum.at[pl.ds(r0, rpt)],
                    sums_out.at[c_ax].at[pl.ds(r0, rpt)])
    pltpu.sync_copy(scnt.at[pl.ds(r0, rpt)],
                    cnt_out.at[c_ax].at[pl.ds(r0, rpt)])

  return k(hmat, col2d, zsum, zcnt, ones)


def _out_body(x_ref, s0_ref, s1_ref, c0_ref, c1_ref, w1b_ref, b1b_ref,
              w2x_ref, w2m_ref, b2a_ref, w2b_ref, b2b_ref, o_ref):
  s = s0_ref[...] + s1_ref[...]
  cnt = (c0_ref[...] + c1_ref[...])[:, :1]
  m = s / jnp.maximum(cnt, 1.0)
  mask = (cnt > 0).astype(jnp.float32)
  mean = (
      jnp.dot(m, w1b_ref[...], preferred_element_type=jnp.float32)
      + b1b_ref[...] * mask
  )
  t = (
      jnp.dot(x_ref[...], w2x_ref[...], preferred_element_type=jnp.float32)
      + jnp.dot(mean, w2m_ref[...], preferred_element_type=jnp.float32)
      + b2a_ref[...]
  )
  t = jnp.where(t > 0, t, jnp.exp(jnp.minimum(t, 0.0)) - 1.0)
  o_ref[...] = (
      jnp.dot(t, w2b_ref[...], preferred_element_type=jnp.float32)
      + b2b_ref[...]
  )


def _out_call(x, s0, s1, c0, c1, w1b, b1b, w2x, w2m, b2a, w2b, b2b):
  n, df = x.shape
  h = s0.shape[1]
  out = w2b.shape[1]
  bn = 2000
  full = lambda a: pl.BlockSpec(a.shape, lambda i: (0,) * a.ndim)
  return pl.pallas_call(
      _out_body,
      grid=(n // bn,),
      in_specs=[
          pl.BlockSpec((bn, df), lambda i: (i, 0)),
          pl.BlockSpec((bn, h), lambda i: (i, 0)),
          pl.BlockSpec((bn, h), lambda i: (i, 0)),
          pl.BlockSpec((bn, 16), lambda i: (i, 0)),
          pl.BlockSpec((bn, 16), lambda i: (i, 0)),
          full(w1b), full(b1b), full(w2x), full(w2m), full(b2a),
          full(w2b), full(b2b),
      ],
      out_specs=pl.BlockSpec((bn, out), lambda i: (i, 0)),
      out_shape=jax.ShapeDtypeStruct((n, out), jnp.float32),
  )(x, s0, s1, c0, c1, w1b, b1b, w2x, w2m, b2a, w2b, b2b)


def kernel(x, edge_index, edge_attr, u, batch,
           W1a, b1a, W1b, b1b, W2a, b2a, W2b, b2b):
  n, df = x.shape
  e, de = edge_attr.shape
  h = W1b.shape[0]
  nchunk = e // CH

  row_flat = edge_index[0].astype(jnp.int32)
  col2d = edge_index[1].astype(jnp.int32).reshape(nchunk, CH)

  w1ax128 = jnp.pad(W1a[:df], ((0, 0), (0, 128 - h)))
  b1a128 = jnp.pad(b1a, (0, 128 - h)).reshape(1, 128)
  xa128 = _xa_call(x, w1ax128, b1a128)
  gx = _gather_call(xa128, row_flat, h)
  hmat = _h_call(gx, edge_attr, W1a[df:])

  zsum = jnp.zeros((n, h), jnp.float32)
  zcnt = jnp.zeros((n, 16), jnp.float32)
  ones = jnp.ones((CH, 16), jnp.float32)
  sums_p, cnt_p = _scatter_call(hmat, col2d, zsum, zcnt, ones)

  return _out_call(
      x, sums_p[0], sums_p[1], cnt_p[0], cnt_p[1],
      W1b, b1b.reshape(1, h), W2a[:df], W2a[df:],
      b2a.reshape(1, h), W2b, b2b.reshape(1, W2b.shape[1]),
  )
